# baseline (speedup 1.0000x reference)
import functools

import jax
import jax.numpy as jnp
from jax import lax
from jax.experimental import pallas as pl
from jax.experimental.pallas import tpu as pltpu
from jax.experimental.pallas import tpu_sc as plsc

SMALL = -1000.0
B, T, D, L = 128, 512, 256, 16
T_BLK = 32
N_BLK = T // T_BLK
NORM_EVERY = 16
CHUNK = 16
N_CHUNK = T // CHUNK


def _crf_body(x_ref, seqlen_ref, wt_ref, b_ref, trans_ref,
              out_ref, pred_out_ref,
              alpha_ref, scale_ref, logz_ref):
    g = pl.program_id(0)
    t_base = g * T_BLK

    seq_len = seqlen_ref[...]
    trans = trans_ref[...]
    exp_t = jnp.exp(trans)

    @pl.when(g == 0)
    def _init():
        lane = jax.lax.broadcasted_iota(jnp.int32, (B, L), 1)
        alpha_ref[...] = (lane == (L - 2)).astype(jnp.float32)
        scale_ref[...] = jnp.zeros((B, 1), jnp.float32)
        logz_ref[...] = jnp.zeros((B, 1), jnp.float32)

    x2d = x_ref[...].reshape(B * T_BLK, D)
    pred2d = jnp.dot(x2d, wt_ref[...], preferred_element_type=jnp.float32)
    pred2d = pred2d + b_ref[...]
    lane2d = jax.lax.broadcasted_iota(jnp.int32, (B * T_BLK, L), 1)
    pred2d = jnp.where(lane2d >= L - 2, SMALL, pred2d)
    pred3 = pred2d.reshape(B, T_BLK, L)
    pred_out_ref[...] = pred3

    alpha = alpha_ref[...]
    scale = scale_ref[...]
    eexp3 = jnp.exp(pred3)

    cap_cols = []
    nrms = []
    for i in range(T_BLK):
        a1 = jnp.dot(alpha, exp_t, preferred_element_type=jnp.float32)
        cap_cols.append(a1[:, L - 1:L])
        alpha = a1 * eexp3[:, i, :]
        if i % NORM_EVERY == NORM_EVERY - 1:
            nrm = jnp.max(alpha, axis=1, keepdims=True)
            alpha = alpha * (1.0 / nrm)
            nrms.append(nrm)

    t_idx = t_base + jax.lax.broadcasted_iota(jnp.int32, (B, T_BLK), 1)
    caps_raw = jnp.concatenate(cap_cols, axis=1)
    lane32 = jax.lax.broadcasted_iota(jnp.int32, (B, T_BLK), 1)
    lognrm8 = jnp.log(jnp.concatenate(nrms, axis=1))
    grpadj = jnp.zeros((B, T_BLK), jnp.float32)
    for gi in range(len(nrms) - 1):
        boundary = (gi + 1) * NORM_EVERY
        grpadj = grpadj + jnp.where(lane32 >= boundary,
                                    lognrm8[:, gi:gi + 1], 0.0)
    caps = scale + grpadj + jnp.log(caps_raw)
    logz_ref[...] = logz_ref[...] + jnp.sum(
        jnp.where(t_idx == seq_len, caps, 0.0), axis=1, keepdims=True)

    alpha_ref[...] = alpha
    scale_ref[...] = scale + jnp.sum(lognrm8, axis=1, keepdims=True)

    @pl.when(g == N_BLK - 1)
    def _fin():
        corr = jnp.where(seq_len == 0, trans[L - 2, L - 1], 0.0)
        out_ref[...] = jnp.sum(logz_ref[...] - corr, keepdims=True)


def _real_path_sc(pred_hbm, tags_hbm, ptags_hbm, slen_hbm, trans_hbm,
                  out_hbm,
                  pred_v, tags_v, ptags_v, slen_v, trans_v, acc_v):
    info = plsc.get_sparse_core_info()
    nw = info.num_cores * info.num_subcores
    rows = B // nw
    wid = lax.axis_index("s") * info.num_cores + lax.axis_index("c")

    pltpu.sync_copy(trans_hbm, trans_v)

    for bi in range(rows):
        b = wid * rows + bi
        pltpu.sync_copy(pred_hbm.at[b], pred_v)
        pltpu.sync_copy(tags_hbm.at[b], tags_v)
        pltpu.sync_copy(ptags_hbm.at[b], ptags_v)
        pltpu.sync_copy(slen_hbm.at[b], slen_v)
        s_vec = slen_v[...]

        def chunk_body(c, acc):
            t_loc = lax.iota(jnp.int32, CHUNK) + c * CHUNK
            tg = tags_v[pl.ds(c * CHUNK, CHUNK)]
            pt = ptags_v[pl.ds(c * CHUNK, CHUNK)]
            em = plsc.load_gather(pred_v, [t_loc, tg])
            tr = plsc.load_gather(trans_v, [pt * L + tg])
            te = plsc.load_gather(trans_v, [tg * L + (L - 1)])
            acc = acc + jnp.where(t_loc < s_vec, em + tr, 0.0)
            acc = acc + jnp.where(t_loc == s_vec - 1, te, 0.0)
            return acc

        acc = lax.fori_loop(0, N_CHUNK, chunk_body,
                            jnp.zeros((CHUNK,), jnp.float32))
        acc_v[...] = acc
        pltpu.sync_copy(acc_v, out_hbm.at[b])


@functools.partial(jax.jit, static_argnames=())
def kernel(input, tags, seq_len, W, b, transitions):
    seqlen2 = seq_len.reshape(B, 1).astype(jnp.int32)
    wt = W.astype(jnp.float32).T
    b2 = b.reshape(1, L).astype(jnp.float32)

    tc_out, pred = pl.pallas_call(
        _crf_body,
        grid=(N_BLK,),
        in_specs=[
            pl.BlockSpec((B, T_BLK, D), lambda g: (0, g, 0)),
            pl.BlockSpec((B, 1), lambda g: (0, 0)),
            pl.BlockSpec((D, L), lambda g: (0, 0)),
            pl.BlockSpec((1, L), lambda g: (0, 0)),
            pl.BlockSpec((L, L), lambda g: (0, 0)),
        ],
        out_specs=[
            pl.BlockSpec((1, 1), lambda g: (0, 0)),
            pl.BlockSpec((B, T_BLK, L), lambda g: (0, g, 0)),
        ],
        out_shape=[
            jax.ShapeDtypeStruct((1, 1), jnp.float32),
            jax.ShapeDtypeStruct((B, T, L), jnp.float32),
        ],
        scratch_shapes=[
            pltpu.VMEM((B, L), jnp.float32),
            pltpu.VMEM((B, 1), jnp.float32),
            pltpu.VMEM((B, 1), jnp.float32),
        ],
    )(input, seqlen2, wt, b2, transitions)

    mesh = plsc.VectorSubcoreMesh(core_axis_name="c", subcore_axis_name="s")
    sc_fn = functools.partial(
        pl.kernel, mesh=mesh,
        compiler_params=pltpu.CompilerParams(needs_layout_passes=False),
        out_type=jax.ShapeDtypeStruct((B, CHUNK), jnp.float32),
        scratch_types=[
            pltpu.VMEM((T, L), jnp.float32),
            pltpu.VMEM((T,), jnp.int32),
            pltpu.VMEM((T,), jnp.int32),
            pltpu.VMEM((CHUNK,), jnp.int32),
            pltpu.VMEM((L * L,), jnp.float32),
            pltpu.VMEM((CHUNK,), jnp.float32),
        ],
    )(_real_path_sc)
    tags_i = tags.astype(jnp.int32)
    ptags = jnp.concatenate(
        [jnp.full((B, 1), L - 2, jnp.int32), tags_i[:, :T - 1]], axis=1)
    slen_b = jnp.broadcast_to(seq_len.astype(jnp.int32)[:, None], (B, CHUNK))
    real_parts = sc_fn(pred, tags_i, ptags, slen_b,
                       transitions.reshape(L * L))

    return tc_out[0, 0] - jnp.sum(real_parts)


# speedup vs baseline: 1.8867x; 1.8867x over previous
import functools

import jax
import jax.numpy as jnp
from jax.experimental import pallas as pl
from jax.experimental.pallas import tpu as pltpu

SMALL = -1000.0
B, T, D, L = 128, 512, 256, 16
T_BLK = 32
N_BLK = T // T_BLK
NORM_EVERY = 16


def _crf_body(x_ref, tags_ref, seqlen_ref, slt_ref, wt_ref, b_ref, trans_ref,
              out_ref,
              alpha_ref, scale_ref, logz_ref, real_ref, carry_ref):
    g = pl.program_id(0)
    t_base = g * T_BLK

    seq_len = seqlen_ref[...]
    seq_len_t = slt_ref[...]
    trans = trans_ref[...]
    exp_t = jnp.exp(trans)

    @pl.when(g == 0)
    def _init():
        sub = jax.lax.broadcasted_iota(jnp.int32, (L, B), 0)
        alpha_ref[...] = (sub == (L - 2)).astype(jnp.float32)
        scale_ref[...] = jnp.zeros((1, B), jnp.float32)
        logz_ref[...] = jnp.zeros((1, B), jnp.float32)
        real_ref[...] = jnp.zeros((B, 1), jnp.float32)
        carry_ref[...] = jnp.full((B, 1), L - 2, jnp.int32)

    x2d = x_ref[...].reshape(B * T_BLK, D)
    pred2d = jnp.dot(x2d, wt_ref[...], preferred_element_type=jnp.float32)
    pred2d = pred2d + b_ref[...]
    lane2d = jax.lax.broadcasted_iota(jnp.int32, (B * T_BLK, L), 1)
    pred2d = jnp.where(lane2d >= L - 2, SMALL, pred2d)
    pred3 = pred2d.reshape(B, T_BLK, L)

    tags = jnp.transpose(tags_ref[...]).astype(jnp.int32)
    lane3 = jax.lax.broadcasted_iota(jnp.int32, (B, T_BLK, L), 2)
    oh_cur = (lane3 == tags[:, :, None]).astype(jnp.float32)
    ptags = jnp.concatenate([carry_ref[...], tags[:, :T_BLK - 1]], axis=1)
    oh_prev = (lane3 == ptags[:, :, None]).astype(jnp.float32)
    carry_ref[...] = tags[:, T_BLK - 1:]

    emit_g = jnp.sum(pred3 * oh_cur, axis=2)
    rowvals = jnp.dot(oh_prev.reshape(B * T_BLK, L), trans,
                      preferred_element_type=jnp.float32).reshape(B, T_BLK, L)
    trans_g = jnp.sum(rowvals * oh_cur, axis=2)
    to_end = jnp.sum(oh_cur * trans[:, L - 1][None, None, :], axis=2)

    bt_idx = t_base + jax.lax.broadcasted_iota(jnp.int32, (B, T_BLK), 1)
    in_seq = (bt_idx < seq_len).astype(jnp.float32)
    at_last = (bt_idx == seq_len - 1).astype(jnp.float32)
    tile_real = jnp.sum(in_seq * (emit_g + trans_g) + at_last * to_end,
                        axis=1, keepdims=True)
    real_ref[...] = real_ref[...] + tile_real

    eye = (jax.lax.broadcasted_iota(jnp.int32, (L, L), 0)
           == jax.lax.broadcasted_iota(jnp.int32, (L, L), 1)
           ).astype(jnp.float32)
    rot_c = []
    for k in range(L):
        rk = pltpu.roll(exp_t, L - k, axis=0) if k else exp_t
        diag = jnp.sum(rk * eye, axis=1, keepdims=True)
        rot_c.append(jnp.broadcast_to(diag, (L, B)))

    eexp = [jnp.exp(jnp.transpose(pred3[:, i, :])) for i in range(T_BLK)]

    alpha = alpha_ref[...]
    scale = scale_ref[...]

    cap_rows = []
    nrms = []
    for i in range(T_BLK):
        terms = [(pltpu.roll(alpha, L - k, axis=0) if k else alpha) * rot_c[k]
                 for k in range(L)]
        while len(terms) > 1:
            terms = [terms[j] + terms[j + 1] for j in range(0, len(terms), 2)]
        a1 = terms[0]
        cap_rows.append(a1[L - 1:L, :])
        alpha = a1 * eexp[i]
        if i % NORM_EVERY == NORM_EVERY - 1:
            nrm = jnp.max(alpha, axis=0, keepdims=True)
            alpha = alpha * (1.0 / nrm)
            nrms.append(nrm)

    t_idx = t_base + jax.lax.broadcasted_iota(jnp.int32, (T_BLK, B), 0)
    caps_raw = jnp.concatenate(cap_rows, axis=0)
    sub32 = jax.lax.broadcasted_iota(jnp.int32, (T_BLK, B), 0)
    lognrm = jnp.log(jnp.concatenate(nrms, axis=0))
    grpadj = jnp.zeros((T_BLK, B), jnp.float32)
    for gi in range(len(nrms) - 1):
        boundary = (gi + 1) * NORM_EVERY
        grpadj = grpadj + jnp.where(sub32 >= boundary,
                                    lognrm[gi:gi + 1, :], 0.0)
    caps = scale + grpadj + jnp.log(caps_raw)
    logz_ref[...] = logz_ref[...] + jnp.sum(
        jnp.where(t_idx == seq_len_t, caps, 0.0), axis=0, keepdims=True)

    alpha_ref[...] = alpha
    scale_ref[...] = scale + jnp.sum(lognrm, axis=0, keepdims=True)

    @pl.when(g == N_BLK - 1)
    def _fin():
        corr = jnp.where(seq_len == 0, trans[L - 2, L - 1], 0.0)
        out_ref[...] = (jnp.sum(logz_ref[...], keepdims=True)
                        - jnp.sum(real_ref[...] + corr, keepdims=True))


@functools.partial(jax.jit, static_argnames=())
def kernel(input, tags, seq_len, W, b, transitions):
    tags_t = tags.T.astype(jnp.float32)
    seqlen2 = seq_len.reshape(B, 1).astype(jnp.int32)
    seqlen_t = seq_len.reshape(1, B).astype(jnp.int32)
    wt = W.astype(jnp.float32).T
    b2 = b.reshape(1, L).astype(jnp.float32)

    out = pl.pallas_call(
        _crf_body,
        grid=(N_BLK,),
        in_specs=[
            pl.BlockSpec((B, T_BLK, D), lambda g: (0, g, 0)),
            pl.BlockSpec((T_BLK, B), lambda g: (g, 0)),
            pl.BlockSpec((B, 1), lambda g: (0, 0)),
            pl.BlockSpec((1, B), lambda g: (0, 0)),
            pl.BlockSpec((D, L), lambda g: (0, 0)),
            pl.BlockSpec((1, L), lambda g: (0, 0)),
            pl.BlockSpec((L, L), lambda g: (0, 0)),
        ],
        out_specs=pl.BlockSpec((1, 1), lambda g: (0, 0)),
        out_shape=jax.ShapeDtypeStruct((1, 1), jnp.float32),
        scratch_shapes=[
            pltpu.VMEM((L, B), jnp.float32),
            pltpu.VMEM((1, B), jnp.float32),
            pltpu.VMEM((1, B), jnp.float32),
            pltpu.VMEM((B, 1), jnp.float32),
            pltpu.VMEM((B, 1), jnp.int32),
        ],
    )(input, tags_t, seqlen2, seqlen_t, wt, b2, transitions)
    return out[0, 0]


# speedup vs baseline: 3.2393x; 1.7169x over previous
import functools

import jax
import jax.numpy as jnp
from jax.experimental import pallas as pl
from jax.experimental.pallas import tpu as pltpu

SMALL = -1000.0
B, T, D, L = 128, 512, 256, 16
T_BLK = 32
N_BLK = T // T_BLK
NORM_EVERY = 16


def _sub_reduce(x):
    s = x
    for sh in (8, 4, 2, 1):
        s = s + pltpu.roll(s, sh, axis=0)
    return s[0:1, :]


def _crf_body(x_ref, tags_ref, slt_ref, wt_ref, b_ref, trans_ref,
              out_ref,
              alpha_ref, scale_ref, logz_ref, real_ref, carry_ref):
    g = pl.program_id(0)
    t_base = g * T_BLK

    seq_len_t = slt_ref[...]
    trans = trans_ref[...]
    exp_t = jnp.exp(trans)
    sub_lb = jax.lax.broadcasted_iota(jnp.int32, (L, B), 0)

    @pl.when(g == 0)
    def _init():
        start_oh = (sub_lb == (L - 2)).astype(jnp.float32)
        alpha_ref[...] = start_oh
        carry_ref[...] = start_oh
        scale_ref[...] = jnp.zeros((1, B), jnp.float32)
        logz_ref[...] = jnp.zeros((1, B), jnp.float32)
        real_ref[...] = jnp.zeros((1, B), jnp.float32)

    x2d = x_ref[...].reshape(B * T_BLK, D)
    pred2d = jnp.dot(x2d, wt_ref[...], preferred_element_type=jnp.float32)
    pred2d = pred2d + b_ref[...]
    lane2d = jax.lax.broadcasted_iota(jnp.int32, (B * T_BLK, L), 1)
    pred2d = jnp.where(lane2d >= L - 2, SMALL, pred2d)
    pred3 = pred2d.reshape(B, T_BLK, L)

    pred_t = [jnp.transpose(pred3[:, i, :]) for i in range(T_BLK)]
    eexp = [jnp.exp(p) for p in pred_t]

    tags_f = tags_ref[...]
    sub_f = sub_lb.astype(jnp.float32)
    trans_tt = jnp.transpose(trans)
    c15 = jnp.broadcast_to(trans[:, L - 1:L], (L, B))
    real_acc = jnp.zeros((1, B), jnp.float32)
    ohp = carry_ref[...]
    for i in range(T_BLK):
        t = t_base + i
        ohc = (sub_f == tags_f[i:i + 1, :]).astype(jnp.float32)
        m = jnp.dot(trans_tt, ohp, preferred_element_type=jnp.float32)
        both = _sub_reduce((pred_t[i] + m) * ohc)
        te = _sub_reduce(ohc * c15)
        in_seq = (seq_len_t > t).astype(jnp.float32)
        at_last = (seq_len_t == t + 1).astype(jnp.float32)
        real_acc = real_acc + in_seq * both + at_last * te
        ohp = ohc
    carry_ref[...] = ohp
    real_ref[...] = real_ref[...] + real_acc

    eye = (jax.lax.broadcasted_iota(jnp.int32, (L, L), 0)
           == jax.lax.broadcasted_iota(jnp.int32, (L, L), 1)
           ).astype(jnp.float32)
    rot_c = []
    for k in range(L):
        rk = pltpu.roll(exp_t, L - k, axis=0) if k else exp_t
        diag = jnp.sum(rk * eye, axis=1, keepdims=True)
        rot_c.append(jnp.broadcast_to(diag, (L, B)))

    alpha = alpha_ref[...]
    scale = scale_ref[...]

    cap_rows = []
    nrms = []
    for i in range(T_BLK):
        terms = [(pltpu.roll(alpha, L - k, axis=0) if k else alpha) * rot_c[k]
                 for k in range(L)]
        while len(terms) > 1:
            terms = [terms[j] + terms[j + 1] for j in range(0, len(terms), 2)]
        a1 = terms[0]
        cap_rows.append(a1[L - 1:L, :])
        alpha = a1 * eexp[i]
        if i % NORM_EVERY == NORM_EVERY - 1:
            nrm = jnp.max(alpha, axis=0, keepdims=True)
            alpha = alpha * (1.0 / nrm)
            nrms.append(nrm)

    t_idx = t_base + jax.lax.broadcasted_iota(jnp.int32, (T_BLK, B), 0)
    caps_raw = jnp.concatenate(cap_rows, axis=0)
    sub32 = jax.lax.broadcasted_iota(jnp.int32, (T_BLK, B), 0)
    lognrm = jnp.log(jnp.concatenate(nrms, axis=0))
    grpadj = jnp.zeros((T_BLK, B), jnp.float32)
    for gi in range(len(nrms) - 1):
        boundary = (gi + 1) * NORM_EVERY
        grpadj = grpadj + jnp.where(sub32 >= boundary,
                                    lognrm[gi:gi + 1, :], 0.0)
    caps = scale + grpadj + jnp.log(caps_raw)
    logz_ref[...] = logz_ref[...] + jnp.sum(
        jnp.where(t_idx == seq_len_t, caps, 0.0), axis=0, keepdims=True)

    alpha_ref[...] = alpha
    scale_ref[...] = scale + jnp.sum(lognrm, axis=0, keepdims=True)

    @pl.when(g == N_BLK - 1)
    def _fin():
        corr = jnp.where(seq_len_t == 0, trans[L - 2, L - 1], 0.0)
        out_ref[...] = jnp.sum(logz_ref[...] - real_ref[...] - corr,
                               keepdims=True)


@functools.partial(jax.jit, static_argnames=())
def kernel(input, tags, seq_len, W, b, transitions):
    tags_t = tags.T.astype(jnp.float32)
    seqlen_t = seq_len.reshape(1, B).astype(jnp.int32)
    wt = W.astype(jnp.float32).T
    b2 = b.reshape(1, L).astype(jnp.float32)

    out = pl.pallas_call(
        _crf_body,
        grid=(N_BLK,),
        in_specs=[
            pl.BlockSpec((B, T_BLK, D), lambda g: (0, g, 0)),
            pl.BlockSpec((T_BLK, B), lambda g: (g, 0)),
            pl.BlockSpec((1, B), lambda g: (0, 0)),
            pl.BlockSpec((D, L), lambda g: (0, 0)),
            pl.BlockSpec((1, L), lambda g: (0, 0)),
            pl.BlockSpec((L, L), lambda g: (0, 0)),
        ],
        out_specs=pl.BlockSpec((1, 1), lambda g: (0, 0)),
        out_shape=jax.ShapeDtypeStruct((1, 1), jnp.float32),
        scratch_shapes=[
            pltpu.VMEM((L, B), jnp.float32),
            pltpu.VMEM((1, B), jnp.float32),
            pltpu.VMEM((1, B), jnp.float32),
            pltpu.VMEM((1, B), jnp.float32),
            pltpu.VMEM((L, B), jnp.float32),
        ],
    )(input, tags_t, seqlen_t, wt, b2, transitions)
    return out[0, 0]
